# Initial kernel scaffold; baseline (speedup 1.0000x reference)
import jax
import jax.numpy as jnp
from jax.experimental import pallas as pl


def kernel(adj, x, W1_0, W1_1, W2_0, W2_1, PredW, PredB):
    raise NotImplementedError("write your pallas kernel here")


# speedup vs baseline: 4.5566x; 4.5566x over previous
import functools

import jax
import jax.numpy as jnp
from jax import lax
from jax.experimental import pallas as pl
from jax.experimental.pallas import tpu as pltpu

BIGNEG = -9e15


def _elu(t):
    return jnp.where(t > 0, t, jnp.exp(jnp.minimum(t, 0.0)) - 1.0)


def _sums_body(a_ref, rs_ref, cs_ref):
    i = pl.program_id(0)
    a = a_ref[...]
    rs_ref[...] = jnp.sum(a, axis=1, keepdims=True) + 1.0
    part = jnp.sum(a, axis=0, keepdims=True)

    @pl.when(i == 0)
    def _():
        cs_ref[...] = part + 1.0

    @pl.when(i > 0)
    def _():
        cs_ref[...] += part


def _sums(adj):
    n = adj.shape[0]
    blk = 512
    return pl.pallas_call(
        _sums_body,
        grid=(n // blk,),
        in_specs=[pl.BlockSpec((blk, n), lambda i: (i, 0))],
        out_specs=[
            pl.BlockSpec((blk, 1), lambda i: (i, 0)),
            pl.BlockSpec((1, n), lambda i: (0, 0)),
        ],
        out_shape=[
            jax.ShapeDtypeStruct((n, 1), jnp.float32),
            jax.ShapeDtypeStruct((1, n), jnp.float32),
        ],
    )(adj)


def _scale_mm_body(z_ref, w_ref, d_ref, o_ref):
    t = jnp.dot(z_ref[...], w_ref[...], preferred_element_type=jnp.float32, precision=jax.lax.Precision.HIGHEST)
    o_ref[...] = t * lax.rsqrt(d_ref[...])


def _scale_mm(z, w, d):
    n = z.shape[0]
    h = w.shape[1]
    return pl.pallas_call(
        _scale_mm_body,
        out_shape=jax.ShapeDtypeStruct((n, h), jnp.float32),
    )(z, w, d)


def _adj_mm_body(a_ref, y_ref, yb_ref, rs_ref, o_ref):
    t = jnp.dot(a_ref[...], y_ref[...], preferred_element_type=jnp.float32, precision=jax.lax.Precision.HIGHEST)
    t = (t + yb_ref[...]) * lax.rsqrt(rs_ref[...])
    o_ref[...] = _elu(t)


def _adj_mm(a, y, rs):
    n, h = y.shape
    blk = 512
    return pl.pallas_call(
        _adj_mm_body,
        grid=(n // blk,),
        in_specs=[
            pl.BlockSpec((blk, n), lambda i: (i, 0)),
            pl.BlockSpec((n, h), lambda i: (0, 0)),
            pl.BlockSpec((blk, h), lambda i: (i, 0)),
            pl.BlockSpec((blk, 1), lambda i: (i, 0)),
        ],
        out_specs=pl.BlockSpec((blk, h), lambda i: (i, 0)),
        out_shape=jax.ShapeDtypeStruct((n, h), jnp.float32),
    )(a, y, y, rs)


def _rownorm_body(x_ref, o_ref):
    x = x_ref[...]
    o_ref[...] = jnp.sqrt(jnp.sum(x * x, axis=1, keepdims=True))


def _rownorm(x):
    return pl.pallas_call(
        _rownorm_body,
        out_shape=jax.ShapeDtypeStruct((x.shape[0], 1), jnp.float32),
    )(x)


def _topk_body(k, xb_ref, xf_ref, lnb_ref, lnr_ref, g_ref, cc_ref):
    i = pl.program_id(0)
    r, n = g_ref.shape
    dot = lax.dot_general(
        xb_ref[...], xf_ref[...], (((1,), (1,)), ((), ())),
        preferred_element_type=jnp.float32,
    )
    s = dot / (lnb_ref[...] * lnr_ref[...])
    col = lax.broadcasted_iota(jnp.int32, (r, n), 1)
    rowg = lax.broadcasted_iota(jnp.int32, (r, n), 0) + i * r
    s = jnp.where(col == rowg, BIGNEG, s)
    oh = jnp.zeros((r, n), jnp.float32)
    for _ in range(k):
        m = jnp.max(s, axis=1, keepdims=True)
        cand = jnp.where(s >= m, col, n)
        idx = jnp.min(cand, axis=1, keepdims=True)
        hit = col == idx
        oh += hit.astype(jnp.float32)
        s = jnp.where(hit, BIGNEG, s)
    g_ref[...] = oh
    part = jnp.sum(oh, axis=0, keepdims=True)

    @pl.when(i == 0)
    def _():
        cc_ref[...] = part + 1.0

    @pl.when(i > 0)
    def _():
        cc_ref[...] += part


def _topk_graph(x, ln_col, ln_row, k):
    n, d = x.shape
    r = 256
    return pl.pallas_call(
        functools.partial(_topk_body, k),
        grid=(n // r,),
        in_specs=[
            pl.BlockSpec((r, d), lambda i: (i, 0)),
            pl.BlockSpec((n, d), lambda i: (0, 0)),
            pl.BlockSpec((r, 1), lambda i: (i, 0)),
            pl.BlockSpec((1, n), lambda i: (0, 0)),
        ],
        out_specs=[
            pl.BlockSpec((r, n), lambda i: (i, 0)),
            pl.BlockSpec((1, n), lambda i: (0, 0)),
        ],
        out_shape=[
            jax.ShapeDtypeStruct((n, n), jnp.float32),
            jax.ShapeDtypeStruct((1, n), jnp.float32),
        ],
    )(x, x, ln_col, ln_row)


def _pred_t_body(h1_ref, h2_ref, wa_ref, wb_ref, o_ref):
    o_ref[...] = (
        jnp.dot(h1_ref[...], wa_ref[...], preferred_element_type=jnp.float32, precision=jax.lax.Precision.HIGHEST)
        + jnp.dot(h2_ref[...], wb_ref[...], preferred_element_type=jnp.float32, precision=jax.lax.Precision.HIGHEST)
    )


def _pred_t(h1, h2, wa, wb):
    n = h1.shape[0]
    m = wa.shape[1]
    return pl.pallas_call(
        _pred_t_body,
        out_shape=jax.ShapeDtypeStruct((n, m), jnp.float32),
    )(h1, h2, wa, wb)


def _logits_body(t1_ref, t2_ref, h1_ref, h2_ref, b_ref, o_ref):
    dn = (((1,), (1,)), ((), ()))
    o_ref[...] = (
        lax.dot_general(t1_ref[...], h1_ref[...], dn,
                        preferred_element_type=jnp.float32, precision=jax.lax.Precision.HIGHEST)
        + lax.dot_general(t2_ref[...], h2_ref[...], dn,
                          preferred_element_type=jnp.float32, precision=jax.lax.Precision.HIGHEST)
        + b_ref[...]
    )


def _logits(t, h1, h2, b):
    n, m = t.shape
    hh = m // 2
    blk = 256
    return pl.pallas_call(
        _logits_body,
        grid=(n // blk,),
        in_specs=[
            pl.BlockSpec((blk, hh), lambda i: (i, 0)),
            pl.BlockSpec((blk, hh), lambda i: (i, 1)),
            pl.BlockSpec((n, hh), lambda i: (0, 0)),
            pl.BlockSpec((n, hh), lambda i: (0, 0)),
            pl.BlockSpec((1, 1), lambda i: (0, 0)),
        ],
        out_specs=pl.BlockSpec((blk, n), lambda i: (i, 0)),
        out_shape=jax.ShapeDtypeStruct((n, n), jnp.float32),
    )(t, t, h1, h2, b)


def kernel(adj, x, W1_0, W1_1, W2_0, W2_1, PredW, PredB):
    n = adj.shape[0]
    hh = W1_0.shape[1]
    rs, cs = _sums(adj)
    csT = cs.reshape(n, 1)
    y = _scale_mm(x, W1_0, csT)
    z = _adj_mm(adj, y, rs)
    y = _scale_mm(z, W1_1, csT)
    h1 = _adj_mm(adj, y, rs)

    ln = _rownorm(x)
    g, ccrow = _topk_graph(x, ln, ln.reshape(1, n), 32)
    cc = ccrow.reshape(n, 1)
    rs33 = jnp.full((n, 1), 33.0, jnp.float32)
    y = _scale_mm(x, W2_0, cc)
    z = _adj_mm(g, y, rs33)
    y = _scale_mm(z, W2_1, cc)
    h2 = _adj_mm(g, y, rs33)

    t = _pred_t(h1, h2, PredW[:hh], PredW[hh:])
    return _logits(t, h1, h2, PredB)
